# baseline (speedup 1.0000x reference)
import jax
import jax.numpy as jnp
from jax.experimental import pallas as pl
from jax.experimental.pallas import tpu as pltpu

_INV_T = 100.0
_L, _P, _D = 16, 768, 256
_N = _L * _P
_BI = 128
_NB = _N // _BI
_CORES = 2
_JC = 1024


def _matmul_body(lm_ref, pf_ref, out_ref):
    out_ref[...] = jax.lax.dot_general(
        lm_ref[...], pf_ref[...],
        (((1,), (1,)), ((), ())),
        preferred_element_type=jnp.float32,
    )


def _main_body(k_ref, u_mat_ref, v_row_ref, out_ref):
    c = pl.program_id(0)
    nb = (k_ref[0] + (_BI - 1)) // _BI
    trips = (nb + 1 - c) // _CORES
    jtrips = (k_ref[1] + (_JC - 1)) // _JC

    def body(b, carry):
        num_acc, npos_acc = carry
        blk = b * _CORES + c
        u_row = u_mat_ref[pl.ds(blk, 1), 0, :]
        u_col = jax.lax.transpose(u_row, (1, 0))
        pos_col = jnp.where(u_col > 10.0, 1.0, 0.0)
        pn_col = jnp.where(u_col > 1.5, 1.0, 0.0)
        s_col = u_col - 3.0 * pn_col - 12.0 * pos_col

        def jbody(jc, acc):
            v = v_row_ref[:, pl.ds(jc * _JC, _JC)]
            pn_w = jnp.where(v > 1.5, 1.0, 0.0)
            pp_w = jnp.where(v > 10.0, 1.0, 0.0)
            s_j = v - 3.0 * pn_w - 12.0 * pp_w
            d = jax.nn.sigmoid((s_j - s_col) * _INV_T)
            w = jnp.concatenate([pn_w, pp_w], axis=0)
            return acc + jax.lax.dot_general(
                d, w, (((1,), (1,)), ((), ())),
                preferred_element_type=jnp.float32,
            )

        acc = jax.lax.fori_loop(
            0, jtrips, jbody, jnp.zeros((_BI, 2), jnp.float32))
        rpn = 1.0 + acc[:, 0:1]
        rp = 1.0 + acc[:, 1:2]
        num_acc = num_acc + jnp.sum(pos_col * rp / rpn)
        npos_acc = npos_acc + jnp.sum(pos_col)
        return num_acc, npos_acc

    num, npos = jax.lax.fori_loop(
        0, trips, body, (jnp.float32(0.0), jnp.float32(0.0)))
    lane = jax.lax.broadcasted_iota(jnp.int32, (1, 1, 128), 2)
    out_ref[...] = jnp.where(lane == 0, num, jnp.where(lane == 1, npos, 0.0))


def kernel(landmark_embeddings, patch_features, pos_patches, pos_neg_patches):
    sims = pl.pallas_call(
        _matmul_body,
        out_shape=jax.ShapeDtypeStruct((_L, _P), jnp.float32),
    )(landmark_embeddings, patch_features)

    s_flat = sims.reshape(-1)
    pos_b = pos_patches.reshape(-1)
    pn_b = pos_neg_patches.reshape(-1)
    pos_f = pos_b.astype(jnp.float32)
    pn_f = pn_b.astype(jnp.float32)

    v = s_flat + 3.0 * pn_f + 12.0 * pos_f
    v_sorted = -jax.lax.sort(-v, dimension=0)
    u_mat = v_sorted.reshape(_NB, 1, _BI)
    v_row = v_sorted.reshape(1, _N)
    k_count = jnp.stack(
        [jnp.sum(pos_b), jnp.sum(pn_b)]).astype(jnp.int32)

    out = pl.pallas_call(
        _main_body,
        grid=(_CORES,),
        in_specs=[
            pl.BlockSpec(memory_space=pltpu.SMEM),
            pl.BlockSpec((_NB, 1, _BI), lambda c: (0, 0, 0)),
            pl.BlockSpec((1, _N), lambda c: (0, 0)),
        ],
        out_specs=pl.BlockSpec((1, 1, 128), lambda c: (c, 0, 0)),
        out_shape=jax.ShapeDtypeStruct((_CORES, 1, 128), jnp.float32),
        compiler_params=pltpu.CompilerParams(
            dimension_semantics=("parallel",),
        ),
    )(k_count, u_mat, v_row)

    num = out[0, 0, 0] + out[1, 0, 0]
    npos = out[0, 0, 1] + out[1, 0, 1]
    return -(num / npos)


# speedup vs baseline: 1.1592x; 1.1592x over previous
import jax
import jax.numpy as jnp
from jax.experimental import pallas as pl
from jax.experimental.pallas import tpu as pltpu

_INV_T = 100.0
_L, _P, _D = 16, 768, 256
_N = _L * _P
_BI = 128
_NB = _N // _BI
_CORES = 2
_JC = 1024


def _pack_body(lm_ref, pf_ref, pos_ref, pn_ref, u_ref, cnt_ref):
    sims = jax.lax.dot_general(
        lm_ref[...], pf_ref[...],
        (((1,), (1,)), ((), ())),
        preferred_element_type=jnp.float32,
    )
    pos = jnp.where(pos_ref[...], 1.0, 0.0)
    pn = jnp.where(pn_ref[...], 1.0, 0.0)
    u_ref[...] = -(sims + 3.0 * pn + 12.0 * pos)
    lane = jax.lax.broadcasted_iota(jnp.int32, (1, 128), 1)
    cnt_ref[...] = jnp.where(
        lane == 0, jnp.sum(pos), jnp.where(lane == 1, jnp.sum(pn), 0.0))


def _main_body(k_ref, u_mat_ref, u_row_ref, out_ref):
    c = pl.program_id(0)
    nb = (k_ref[0] + (_BI - 1)) // _BI
    trips = (nb + 1 - c) // _CORES

    def body(b, carry):
        num_acc, npos_acc = carry
        blk = b * _CORES + c
        u_row = u_mat_ref[pl.ds(blk, 1), 0, :]
        u_col = jax.lax.transpose(u_row, (1, 0))
        pos_col = jnp.where(u_col < -10.0, 1.0, 0.0)
        pn_col = jnp.where(u_col < -1.5, 1.0, 0.0)
        s_col = -u_col - 3.0 * pn_col - 12.0 * pos_col

        acc = jnp.zeros((_BI, 2), dtype=jnp.float32)
        for jc in range(_N // _JC):
            v = u_row_ref[:, jc * _JC:(jc + 1) * _JC]
            pn_w = jnp.where(v < -1.5, 1.0, 0.0)
            pp_w = jnp.where(v < -10.0, 1.0, 0.0)
            s_j = -v - 3.0 * pn_w - 12.0 * pp_w
            d = jax.nn.sigmoid((s_j - s_col) * _INV_T)
            w = jnp.concatenate([pn_w, pp_w], axis=0)
            acc = acc + jax.lax.dot_general(
                d, w, (((1,), (1,)), ((), ())),
                preferred_element_type=jnp.float32,
            )
        rpn = 1.0 + acc[:, 0:1]
        rp = 1.0 + acc[:, 1:2]
        num_acc = num_acc + jnp.sum(pos_col * rp / rpn)
        npos_acc = npos_acc + jnp.sum(pos_col)
        return num_acc, npos_acc

    num, npos = jax.lax.fori_loop(
        0, trips, body, (jnp.float32(0.0), jnp.float32(0.0)))
    lane = jax.lax.broadcasted_iota(jnp.int32, (1, 1, 128), 2)
    out_ref[...] = jnp.where(lane == 0, num, jnp.where(lane == 1, npos, 0.0))


def kernel(landmark_embeddings, patch_features, pos_patches, pos_neg_patches):
    u, cnt = pl.pallas_call(
        _pack_body,
        out_shape=(
            jax.ShapeDtypeStruct((_L, _P), jnp.float32),
            jax.ShapeDtypeStruct((1, 128), jnp.float32),
        ),
    )(landmark_embeddings, patch_features, pos_patches, pos_neg_patches)

    u_sorted = jax.lax.sort(u.reshape(-1), dimension=0)
    k_count = cnt[0, :2].astype(jnp.int32)

    out = pl.pallas_call(
        _main_body,
        grid=(_CORES,),
        in_specs=[
            pl.BlockSpec(memory_space=pltpu.SMEM),
            pl.BlockSpec((_NB, 1, _BI), lambda c: (0, 0, 0)),
            pl.BlockSpec((1, _N), lambda c: (0, 0)),
        ],
        out_specs=pl.BlockSpec((1, 1, 128), lambda c: (c, 0, 0)),
        out_shape=jax.ShapeDtypeStruct((_CORES, 1, 128), jnp.float32),
        compiler_params=pltpu.CompilerParams(
            dimension_semantics=("parallel",),
        ),
    )(k_count, u_sorted.reshape(_NB, 1, _BI), u_sorted.reshape(1, _N))

    num = out[0, 0, 0] + out[1, 0, 0]
    npos = out[0, 0, 1] + out[1, 0, 1]
    return -(num / npos)


# speedup vs baseline: 1.5276x; 1.3178x over previous
import jax
import jax.numpy as jnp
from jax.experimental import pallas as pl
from jax.experimental.pallas import tpu as pltpu

_INV_T = 100.0
_L, _P, _D = 16, 768, 256
_N = _L * _P
_BI = 128
_NB = _N // _BI
_JC = 1024


def _pack_body(lm_ref, pf_ref, mm_ref, u_ref, cnt_ref):
    sims = jax.lax.dot_general(
        lm_ref[...], pf_ref[...],
        (((1,), (1,)), ((), ())),
        preferred_element_type=jnp.float32,
    )
    mm = mm_ref[...].astype(jnp.int32)
    pos = jnp.where(mm >= 2, 1.0, 0.0)
    pn = jnp.where((mm & 1) == 1, 1.0, 0.0)
    u_ref[...] = -(sims + 3.0 * pn + 12.0 * pos)
    lane = jax.lax.broadcasted_iota(jnp.int32, (1, 128), 1)
    cnt_ref[...] = jnp.where(
        lane == 0, jnp.sum(pos), jnp.where(lane == 1, jnp.sum(pn), 0.0))


def _main_body(k_ref, u_mat_ref, u_row_ref, out_ref):
    nb = (k_ref[0] + (_BI - 1)) // _BI

    def body(blk, carry):
        num_acc, npos_acc = carry
        u_row = u_mat_ref[pl.ds(blk, 1), 0, :]
        u_col = jax.lax.transpose(u_row, (1, 0))
        pos_col = jnp.where(u_col < -10.0, 1.0, 0.0)
        pn_col = jnp.where(u_col < -1.5, 1.0, 0.0)
        s_col = -u_col - 3.0 * pn_col - 12.0 * pos_col
        sc100 = (s_col * _INV_T).astype(jnp.bfloat16)

        accs = []
        for jc in range(_N // _JC):
            v = u_row_ref[:, jc * _JC:(jc + 1) * _JC]
            m_pn = v < -1.5
            m_pp = v < -10.0
            s_j = (-v - jnp.where(m_pn, 3.0, 0.0)
                   - jnp.where(m_pp, 12.0, 0.0))
            sj100 = (s_j * _INV_T).astype(jnp.bfloat16)
            d = jax.nn.sigmoid(sj100 - sc100)
            w = jnp.concatenate(
                [jnp.where(m_pn, 1.0, 0.0),
                 jnp.where(m_pp, 1.0, 0.0)],
                axis=0).astype(jnp.bfloat16)
            accs.append(jax.lax.dot_general(
                d, w, (((1,), (1,)), ((), ())),
                preferred_element_type=jnp.float32,
            ))
        while len(accs) > 1:
            accs = [a + b for a, b in zip(accs[::2], accs[1::2])]
        acc = accs[0]
        rpn = 1.0 + acc[:, 0:1]
        rp = 1.0 + acc[:, 1:2]
        num_acc = num_acc + jnp.sum(pos_col * rp / rpn)
        npos_acc = npos_acc + jnp.sum(pos_col)
        return num_acc, npos_acc

    num, npos = jax.lax.fori_loop(
        0, nb, body, (jnp.float32(0.0), jnp.float32(0.0)))
    lane = jax.lax.broadcasted_iota(jnp.int32, (1, 128), 1)
    out_ref[...] = jnp.where(lane == 0, -(num / npos), 0.0)


def kernel(landmark_embeddings, patch_features, pos_patches, pos_neg_patches):
    mm = ((pos_patches.astype(jnp.uint8) << 1)
          | pos_neg_patches.astype(jnp.uint8))

    u, cnt = pl.pallas_call(
        _pack_body,
        out_shape=(
            jax.ShapeDtypeStruct((_L, _P), jnp.float32),
            jax.ShapeDtypeStruct((1, 128), jnp.float32),
        ),
    )(landmark_embeddings, patch_features, mm)

    u_sorted = jax.lax.sort(u.reshape(-1), dimension=0)
    k_count = cnt[0, :2].astype(jnp.int32)

    out = pl.pallas_call(
        _main_body,
        in_specs=[
            pl.BlockSpec(memory_space=pltpu.SMEM),
            pl.BlockSpec((_NB, 1, _BI), lambda: (0, 0, 0)),
            pl.BlockSpec((1, _N), lambda: (0, 0)),
        ],
        out_specs=pl.BlockSpec((1, 128), lambda: (0, 0)),
        out_shape=jax.ShapeDtypeStruct((1, 128), jnp.float32),
    )(k_count, u_sorted.reshape(_NB, 1, _BI), u_sorted.reshape(1, _N))

    return out[0, 0]


# speedup vs baseline: 1.7455x; 1.1426x over previous
import jax
import jax.numpy as jnp
from jax.experimental import pallas as pl
from jax.experimental.pallas import tpu as pltpu

_INV_T = 100.0
_L, _P, _D = 16, 768, 256
_N = _L * _P
_BI = 128
_NB = _N // _BI
_JC = 1024


def _pack_body(lm_ref, pf_ref, mm_ref, u_ref, cnt_ref):
    sims = jax.lax.dot_general(
        lm_ref[...], pf_ref[...],
        (((1,), (1,)), ((), ())),
        preferred_element_type=jnp.float32,
    )
    mm = mm_ref[...].astype(jnp.int32)
    pos = jnp.where(mm >= 2, 1.0, 0.0)
    pn = jnp.where((mm & 1) == 1, 1.0, 0.0)
    u_ref[...] = -(sims + 3.0 * pn + 12.0 * pos)
    lane = jax.lax.broadcasted_iota(jnp.int32, (1, 128), 1)
    cnt_ref[...] = jnp.where(
        lane == 0, jnp.sum(pos), jnp.where(lane == 1, jnp.sum(pn), 0.0))


def _main_body(k_ref, u_mat_ref, u_row_ref, out_ref):
    nb = (k_ref[0] + (_BI - 1)) // _BI
    kpos_f = k_ref[0].astype(jnp.float32)
    kpn_f = k_ref[1].astype(jnp.float32)

    def body(blk, carry):
        num_acc, npos_acc = carry
        u_row = u_mat_ref[pl.ds(blk, 1), 0, :]
        u_col = jax.lax.transpose(u_row, (1, 0))
        pos_col = jnp.where(u_col < -10.0, 1.0, 0.0)
        pn_col = jnp.where(u_col < -1.5, 1.0, 0.0)
        s_col = -u_col - 3.0 * pn_col - 12.0 * pos_col
        sc50 = (s_col * (0.5 * _INV_T)).astype(jnp.bfloat16)

        accs = []
        for jc in range(_N // _JC):
            v = u_row_ref[:, jc * _JC:(jc + 1) * _JC]
            m_pn = v < -1.5
            m_pp = v < -10.0
            s_j = (-v - jnp.where(m_pn, 3.0, 0.0)
                   - jnp.where(m_pp, 12.0, 0.0))
            sj50 = (s_j * (0.5 * _INV_T)).astype(jnp.bfloat16)
            t = jax.lax.tanh(sj50 - sc50)
            w = jnp.concatenate(
                [jnp.where(m_pn, 1.0, 0.0),
                 jnp.where(m_pp, 1.0, 0.0)],
                axis=0).astype(jnp.bfloat16)
            accs.append(jax.lax.dot_general(
                t, w, (((1,), (1,)), ((), ())),
                preferred_element_type=jnp.float32,
            ))
        while len(accs) > 1:
            accs = [a + b for a, b in zip(accs[::2], accs[1::2])]
        acc = accs[0]
        rpn = 1.0 + 0.5 * (kpn_f + acc[:, 0:1])
        rp = 1.0 + 0.5 * (kpos_f + acc[:, 1:2])
        num_acc = num_acc + jnp.sum(pos_col * rp / rpn)
        npos_acc = npos_acc + jnp.sum(pos_col)
        return num_acc, npos_acc

    num, npos = jax.lax.fori_loop(
        0, nb, body, (jnp.float32(0.0), jnp.float32(0.0)))
    lane = jax.lax.broadcasted_iota(jnp.int32, (1, 128), 1)
    out_ref[...] = jnp.where(lane == 0, -(num / npos), 0.0)


def kernel(landmark_embeddings, patch_features, pos_patches, pos_neg_patches):
    mm = ((pos_patches.astype(jnp.uint8) << 1)
          | pos_neg_patches.astype(jnp.uint8))

    u, cnt = pl.pallas_call(
        _pack_body,
        out_shape=(
            jax.ShapeDtypeStruct((_L, _P), jnp.float32),
            jax.ShapeDtypeStruct((1, 128), jnp.float32),
        ),
    )(landmark_embeddings, patch_features, mm)

    u_sorted = jax.lax.sort(u.reshape(-1), dimension=0)
    k_count = cnt[0, :2].astype(jnp.int32)

    out = pl.pallas_call(
        _main_body,
        in_specs=[
            pl.BlockSpec(memory_space=pltpu.SMEM),
            pl.BlockSpec((_NB, 1, _BI), lambda: (0, 0, 0)),
            pl.BlockSpec((1, _N), lambda: (0, 0)),
        ],
        out_specs=pl.BlockSpec((1, 128), lambda: (0, 0)),
        out_shape=jax.ShapeDtypeStruct((1, 128), jnp.float32),
    )(k_count, u_sorted.reshape(_NB, 1, _BI), u_sorted.reshape(1, _N))

    return out[0, 0]
